# baseline (speedup 1.0000x reference)
import functools

import jax
import jax.numpy as jnp
from jax import lax
from jax.experimental import pallas as pl
from jax.experimental.pallas import tpu as pltpu
from jax.experimental.pallas import tpu_sc as plsc

_B, _T, _D, _E, _TOPK = 4, 4096, 4096, 64, 2
_BT = _B * _T
_TM = 1024
_NW = 32
_CT = _BT // _NW
_L = 16


def _gate_softmax_block(x_ref, w_ref, probs_ref):
    x = x_ref[...]
    w = w_ref[...]
    logits = lax.dot_general(x, w, (((1,), (1,)), ((), ())),
                             preferred_element_type=jnp.float32)
    m = jnp.max(logits, axis=-1, keepdims=True)
    ex = jnp.exp(logits - m)
    probs_ref[...] = ex / jnp.sum(ex, axis=-1, keepdims=True)


def _gate_softmax(x2, W):
    nt = x2.shape[0]
    return pl.pallas_call(
        _gate_softmax_block,
        grid=(nt // _TM,),
        in_specs=[
            pl.BlockSpec((_TM, _D), lambda i: (i, 0)),
            pl.BlockSpec((_E, _D), lambda i: (0, 0)),
        ],
        out_specs=pl.BlockSpec((_TM, _E), lambda i: (i, 0)),
        out_shape=jax.ShapeDtypeStruct((nt, _E), jnp.float32),
    )(x2, W)


def _top2_body(ct, probs_hbm, idx_hbm, wts_hbm, p_v, i_v, w_v):
    wid = lax.axis_index("s") * 2 + lax.axis_index("c")
    base = wid * ct
    pltpu.sync_copy(probs_hbm.at[pl.ds(base * _E, ct * _E)], p_v)

    lanes = lax.iota(jnp.int32, _L)
    nstrip = 4
    stripw = _E // nstrip

    def _merge(a, b):
        am1, am2, ai1, ai2 = a
        bm1, bm2, bi1, bi2 = b
        take_b = bm1 > am1
        m1 = jnp.maximum(am1, bm1)
        i1 = jnp.where(take_b, bi1, ai1)
        m2 = jnp.where(take_b, jnp.maximum(am1, bm2), jnp.maximum(am2, bm1))
        i2_a = jnp.where(am2 >= bm1, ai2, bi1)
        i2_b = jnp.where(am1 >= bm2, ai1, bi2)
        i2 = jnp.where(take_b, i2_b, i2_a)
        return m1, m2, i1, i2

    def group_body(g, _):
        tok = g * _L + lanes
        rowbase = tok * _E
        states = []
        for st in range(nstrip):
            m1 = jnp.full((_L,), -1.0, jnp.float32)
            m2 = jnp.full((_L,), -1.0, jnp.float32)
            i1 = jnp.zeros((_L,), jnp.int32)
            i2 = jnp.zeros((_L,), jnp.int32)
            states.append((m1, m2, i1, i2))
        for u in range(stripw):
            new_states = []
            for st in range(nstrip):
                m1, m2, i1, i2 = states[st]
                e = st * stripw + u
                ev = jnp.full((_L,), e, jnp.int32)
                p = plsc.load_gather(p_v, [rowbase + e])
                gt1 = p > m1
                gt2 = p > m2
                i2 = jnp.where(gt2, jnp.where(gt1, i1, ev), i2)
                m2 = jnp.where(gt2, jnp.where(gt1, m1, p), m2)
                i1 = jnp.where(gt1, ev, i1)
                m1 = jnp.where(gt1, p, m1)
                new_states.append((m1, m2, i1, i2))
            states = new_states
        m1, m2, i1, i2 = _merge(_merge(states[0], states[1]),
                                _merge(states[2], states[3]))
        s = m1 + m2
        out2 = tok * _TOPK
        plsc.store_scatter(i_v, [out2], i1)
        plsc.store_scatter(i_v, [out2 + 1], i2)
        plsc.store_scatter(w_v, [out2], m1 / s)
        plsc.store_scatter(w_v, [out2 + 1], m2 / s)
        return 0

    lax.fori_loop(0, ct // _L, group_body, 0)
    pltpu.sync_copy(i_v, idx_hbm.at[pl.ds(base * _TOPK, ct * _TOPK)])
    pltpu.sync_copy(w_v, wts_hbm.at[pl.ds(base * _TOPK, ct * _TOPK)])


def _top2_sc(probs):
    nt = probs.shape[0]
    ct = nt // _NW
    mesh = plsc.VectorSubcoreMesh(core_axis_name="c", subcore_axis_name="s")
    k = functools.partial(
        pl.kernel,
        mesh=mesh,
        out_type=[
            jax.ShapeDtypeStruct((nt * _TOPK,), jnp.int32),
            jax.ShapeDtypeStruct((nt * _TOPK,), jnp.float32),
        ],
        scratch_types=[
            pltpu.VMEM((ct * _E,), jnp.float32),
            pltpu.VMEM((ct * _TOPK,), jnp.int32),
            pltpu.VMEM((ct * _TOPK,), jnp.float32),
        ],
        compiler_params=pltpu.CompilerParams(needs_layout_passes=False),
    )(functools.partial(_top2_body, ct))
    return k(probs.reshape(nt * _E))


_NCHUNK = 2


def kernel(x, W):
    x2 = x.reshape(_BT, _D)
    ch = _BT // _NCHUNK
    probs_c, idx_c, wts_c = [], [], []
    for c in range(_NCHUNK):
        p = _gate_softmax(x2[c * ch:(c + 1) * ch], W)
        i, w = _top2_sc(p)
        probs_c.append(p)
        idx_c.append(i)
        wts_c.append(w)
    probs = jnp.concatenate(probs_c, axis=0)
    idx = jnp.concatenate(idx_c, axis=0)
    wts = jnp.concatenate(wts_c, axis=0)
    return (probs.reshape(_B, _T, _E),
            idx.reshape(_B, _T, _TOPK),
            wts.reshape(_B, _T, _TOPK))


# speedup vs baseline: 2.9130x; 2.9130x over previous
import jax
import jax.numpy as jnp
from jax import lax
from jax.experimental import pallas as pl
from jax.experimental.pallas import tpu as pltpu

_B, _T, _D, _E, _TOPK = 4, 4096, 4096, 64, 2
_TM = 1536


def _router_block(x_ref, w_ref, probs_ref, idx_ref, wts_ref):
    x = x_ref[...]
    w = w_ref[...]
    logits = lax.dot_general(x, w, (((1,), (1,)), ((), ())),
                             preferred_element_type=jnp.float32)
    m = jnp.max(logits, axis=-1, keepdims=True)
    ex = jnp.exp(logits - m)
    probs = ex / jnp.sum(ex, axis=-1, keepdims=True)
    probs_ref[...] = probs

    lane = lax.broadcasted_iota(jnp.int32, probs.shape, 1)
    m1 = jnp.max(probs, axis=-1, keepdims=True)
    i1 = jnp.min(jnp.where(probs == m1, lane, _E), axis=-1, keepdims=True)
    masked = jnp.where(lane == i1, -1.0, probs)
    m2 = jnp.max(masked, axis=-1, keepdims=True)
    i2 = jnp.min(jnp.where(masked == m2, lane, _E), axis=-1, keepdims=True)
    s = m1 + m2
    idx_ref[:, 0:1] = i1
    idx_ref[:, 1:2] = i2
    wts_ref[:, 0:1] = m1 / s
    wts_ref[:, 1:2] = m2 / s


def kernel(x, W):
    BT = _B * _T
    x2 = x.reshape(BT, _D)
    grid = (pl.cdiv(BT, _TM),)
    probs, idx, wts = pl.pallas_call(
        _router_block,
        grid=grid,
        in_specs=[
            pl.BlockSpec((_TM, _D), lambda i: (i, 0)),
            pl.BlockSpec((_E, _D), lambda i: (0, 0)),
        ],
        out_specs=[
            pl.BlockSpec((_TM, _E), lambda i: (i, 0)),
            pl.BlockSpec((_TM, _TOPK), lambda i: (i, 0)),
            pl.BlockSpec((_TM, _TOPK), lambda i: (i, 0)),
        ],
        out_shape=[
            jax.ShapeDtypeStruct((BT, _E), jnp.float32),
            jax.ShapeDtypeStruct((BT, _TOPK), jnp.int32),
            jax.ShapeDtypeStruct((BT, _TOPK), jnp.float32),
        ],
        compiler_params=pltpu.CompilerParams(
            vmem_limit_bytes=128 * 1024 * 1024),
    )(x2, W)
    return (probs.reshape(_B, _T, _E),
            idx.reshape(_B, _T, _TOPK),
            wts.reshape(_B, _T, _TOPK))


# speedup vs baseline: 2.9599x; 1.0161x over previous
import jax
import jax.numpy as jnp
from jax import lax
from jax.experimental import pallas as pl
from jax.experimental.pallas import tpu as pltpu

_B, _T, _D, _E, _TOPK = 4, 4096, 4096, 64, 2
_TM = 1024


def _router_block(xa_ref, xb_ref, w_ref, probs_ref, idx_ref, wts_ref):
    xa = xa_ref[...]
    xb = xb_ref[...]
    w = w_ref[...]
    dn = (((1,), (1,)), ((), ()))
    logits = (lax.dot_general(xa, w[:, :_D // 2], dn,
                              preferred_element_type=jnp.float32)
              + lax.dot_general(xb, w[:, _D // 2:], dn,
                                preferred_element_type=jnp.float32))
    m = jnp.max(logits, axis=-1, keepdims=True)
    ex = jnp.exp(logits - m)
    probs = ex / jnp.sum(ex, axis=-1, keepdims=True)
    probs_ref[...] = probs

    lane = lax.broadcasted_iota(jnp.int32, probs.shape, 1)
    m1 = jnp.max(probs, axis=-1, keepdims=True)
    i1 = jnp.min(jnp.where(probs == m1, lane, _E), axis=-1, keepdims=True)
    masked = jnp.where(lane == i1, -1.0, probs)
    m2 = jnp.max(masked, axis=-1, keepdims=True)
    i2 = jnp.min(jnp.where(masked == m2, lane, _E), axis=-1, keepdims=True)
    s = m1 + m2
    idx_ref[:, 0:1] = i1
    idx_ref[:, 1:2] = i2
    wts_ref[:, 0:1] = m1 / s
    wts_ref[:, 1:2] = m2 / s


def kernel(x, W):
    BT = _B * _T
    x2 = x.reshape(BT, _D)
    grid = (pl.cdiv(BT, _TM),)
    probs, idx, wts = pl.pallas_call(
        _router_block,
        grid=grid,
        in_specs=[
            pl.BlockSpec((_TM, _D // 2), lambda i: (i, 0)),
            pl.BlockSpec((_TM, _D // 2), lambda i: (i, 1)),
            pl.BlockSpec((_E, _D), lambda i: (0, 0)),
        ],
        out_specs=[
            pl.BlockSpec((_TM, _E), lambda i: (i, 0)),
            pl.BlockSpec((_TM, _TOPK), lambda i: (i, 0)),
            pl.BlockSpec((_TM, _TOPK), lambda i: (i, 0)),
        ],
        out_shape=[
            jax.ShapeDtypeStruct((BT, _E), jnp.float32),
            jax.ShapeDtypeStruct((BT, _TOPK), jnp.int32),
            jax.ShapeDtypeStruct((BT, _TOPK), jnp.float32),
        ],
        compiler_params=pltpu.CompilerParams(
            vmem_limit_bytes=128 * 1024 * 1024),
    )(x2, x2, W)
    return (probs.reshape(_B, _T, _E),
            idx.reshape(_B, _T, _TOPK),
            wts.reshape(_B, _T, _TOPK))
